# baseline (speedup 1.0000x reference)
import functools

import jax
import jax.numpy as jnp
from jax import lax
from jax.experimental import pallas as pl
from jax.experimental.pallas import tpu as pltpu
from jax.experimental.pallas import tpu_sc as plsc

H, W = 480, 640
NPIX = H * W
NC, NS = 2, 16
NW = NC * NS
CHUNK = 2048
NGRP = CHUNK // 16
ROWS_PER_TILE = NPIX // NS


def _floor_i(v):
    ti = v.astype(jnp.int32)
    tf = ti.astype(jnp.float32)
    return jnp.where(tf > v, ti - 1, ti)


def _round_half_even_i(v):
    f = _floor_i(v)
    fr = v - f.astype(jnp.float32)
    up = (fr > 0.5) | ((fr == 0.5) & ((f & 1) == 1))
    return jnp.where(up, f + 1, f)


def _make_sc_kernel(n_pad):
    ev_per_tile = n_pad // NW
    nchunk = ev_per_tile // CHUNK
    mesh = plsc.VectorSubcoreMesh(core_axis_name="c", subcore_axis_name="s")

    @functools.partial(
        pl.kernel,
        mesh=mesh,
        out_type=jax.ShapeDtypeStruct((NC, 2, NPIX), jnp.float32),
        scratch_types=[
            pltpu.VMEM((CHUNK,), jnp.float32),
            pltpu.VMEM((CHUNK,), jnp.float32),
            pltpu.VMEM((CHUNK,), jnp.float32),
            pltpu.VMEM((CHUNK,), jnp.float32),
            pltpu.VMEM((CHUNK,), jnp.float32),
            pltpu.VMEM((CHUNK,), jnp.int32),
            pltpu.VMEM((CHUNK,), jnp.float32),
            pltpu.VMEM((CHUNK,), jnp.float32),
            pltpu.VMEM((4 * CHUNK,), jnp.int32),
            pltpu.VMEM((4 * CHUNK,), jnp.float32),
            pltpu.VMEM((4 * CHUNK,), jnp.float32),
            pltpu.VMEM_SHARED((NPIX,), jnp.float32),
            pltpu.VMEM_SHARED((NPIX,), jnp.float32),
        ],
    )
    def body(t_hbm, y_hbm, x_hbm, p0_hbm, p1_hbm, fx_hbm, fy_hbm, z_hbm,
             out_hbm, tbuf, ybuf, xbuf, p0buf, p1buf, fidx, fxb, fyb,
             pidx, valp, valn, spimgp, spimgn):
        c = lax.axis_index("c")
        s = lax.axis_index("s")
        wid = s * NC + c

        pltpu.sync_copy(z_hbm.at[pl.ds(s * ROWS_PER_TILE, ROWS_PER_TILE)],
                        spimgp.at[pl.ds(s * ROWS_PER_TILE, ROWS_PER_TILE)])
        pltpu.sync_copy(z_hbm.at[pl.ds(s * ROWS_PER_TILE, ROWS_PER_TILE)],
                        spimgn.at[pl.ds(s * ROWS_PER_TILE, ROWS_PER_TILE)])
        plsc.subcore_barrier()

        def chunk_body(ci, carry):
            base = wid * ev_per_tile + ci * CHUNK
            pltpu.sync_copy(t_hbm.at[pl.ds(base, CHUNK)], tbuf)
            pltpu.sync_copy(y_hbm.at[pl.ds(base, CHUNK)], ybuf)
            pltpu.sync_copy(x_hbm.at[pl.ds(base, CHUNK)], xbuf)
            pltpu.sync_copy(p0_hbm.at[pl.ds(base, CHUNK)], p0buf)
            pltpu.sync_copy(p1_hbm.at[pl.ds(base, CHUNK)], p1buf)

            @plsc.parallel_loop(0, NGRP, unroll=4)
            def pass1(g):
                off = g * 16
                y = ybuf[pl.ds(off, 16)]
                x = xbuf[pl.ds(off, 16)]
                ry = _round_half_even_i(y)
                rx = _round_half_even_i(x)
                ry = jnp.minimum(jnp.maximum(ry, 0), H - 1)
                rx = jnp.minimum(jnp.maximum(rx, 0), W - 1)
                fidx[pl.ds(off, 16)] = ry * W + rx

            pltpu.sync_copy(fx_hbm.at[fidx], fxb)
            pltpu.sync_copy(fy_hbm.at[fidx], fyb)

            @plsc.parallel_loop(0, NGRP, unroll=2)
            def pass2(g):
                off = g * 16
                t = tbuf[pl.ds(off, 16)]
                y = ybuf[pl.ds(off, 16)]
                x = xbuf[pl.ds(off, 16)]
                w0 = p0buf[pl.ds(off, 16)]
                w1 = p1buf[pl.ds(off, 16)]
                fx = fxb[pl.ds(off, 16)]
                fy = fyb[pl.ds(off, 16)]
                dt = 1.0 - t
                wy = y + dt * fy
                wx = x + dt * fx
                y0 = _floor_i(wy)
                x0 = _floor_i(wx)
                dy = wy - y0.astype(jnp.float32)
                dx = wx - x0.astype(jnp.float32)
                wty = 1.0 - dy
                wlx = 1.0 - dx
                y1 = y0 + 1
                x1 = x0 + 1
                y0ok = (y0 >= 0) & (y0 < H)
                y1ok = (y1 >= 0) & (y1 < H)
                x0ok = (x0 >= 0) & (x0 < W)
                x1ok = (x1 >= 0) & (x1 < W)
                base = y0 * W + x0
                corners = (
                    (y0ok & x0ok, base, wty * wlx),
                    (y0ok & x1ok, base + 1, wty * dx),
                    (y1ok & x0ok, base + W, dy * wlx),
                    (y1ok & x1ok, base + W + 1, dy * dx),
                )
                for j, (inb, praw, wgt) in enumerate(corners):
                    p = jnp.where(inb, praw, 0)
                    wv = jnp.where(inb, wgt, 0.0)
                    pidx[pl.ds(j * CHUNK + off, 16)] = p
                    valp[pl.ds(j * CHUNK + off, 16)] = wv * w0
                    valn[pl.ds(j * CHUNK + off, 16)] = wv * w1

            pltpu.sync_copy(valp, spimgp.at[pidx], add=True)
            pltpu.sync_copy(valn, spimgn.at[pidx], add=True)
            return carry

        lax.fori_loop(0, nchunk, chunk_body, 0)
        plsc.subcore_barrier()

        pltpu.sync_copy(spimgp.at[pl.ds(s * ROWS_PER_TILE, ROWS_PER_TILE)],
                        out_hbm.at[c, 0, pl.ds(s * ROWS_PER_TILE, ROWS_PER_TILE)])
        pltpu.sync_copy(spimgn.at[pl.ds(s * ROWS_PER_TILE, ROWS_PER_TILE)],
                        out_hbm.at[c, 1, pl.ds(s * ROWS_PER_TILE, ROWS_PER_TILE)])

    return body


def kernel(event_list, flow, pol_mask, event_mask):
    n = event_list.shape[1]
    n_pad = ((n + NW * CHUNK - 1) // (NW * CHUNK)) * (NW * CHUNK)
    pad = n_pad - n
    ev = event_list[0]
    t = ev[:, 0]
    y = ev[:, 1]
    x = ev[:, 2]
    p0 = pol_mask[0, :, 0]
    p1 = pol_mask[0, :, 1]
    if pad:
        z = jnp.zeros((pad,), jnp.float32)
        t = jnp.concatenate([t, z])
        y = jnp.concatenate([y, z])
        x = jnp.concatenate([x, z])
        p0 = jnp.concatenate([p0, z])
        p1 = jnp.concatenate([p1, z])
    fx = flow[0, 0].reshape(-1)
    fy = flow[0, 1].reshape(-1)
    zeros1 = jnp.zeros((NPIX,), jnp.float32)
    out = _make_sc_kernel(n_pad)(t, y, x, p0, p1, fx, fy, zeros1)
    acc = out[0] + out[1]
    return acc.reshape(1, 2, H, W)


# speedup vs baseline: 1.3159x; 1.3159x over previous
import functools

import jax
import jax.numpy as jnp
from jax import lax
from jax.experimental import pallas as pl
from jax.experimental.pallas import tpu as pltpu
from jax.experimental.pallas import tpu_sc as plsc

H, W = 480, 640
NPIX = H * W
NC, NS = 2, 16
NW = NC * NS
CHUNK = 2048
NGRP = CHUNK // 16
ROWS_PER_TILE = NPIX // NS


def _floor_i(v):
    ti = v.astype(jnp.int32)
    tf = ti.astype(jnp.float32)
    return jnp.where(tf > v, ti - 1, ti)


def _round_half_even_i(v):
    f = _floor_i(v)
    fr = v - f.astype(jnp.float32)
    up = (fr > 0.5) | ((fr == 0.5) & ((f & 1) == 1))
    return jnp.where(up, f + 1, f)


def _make_sc_kernel(n_pad):
    ev_per_tile = n_pad // NW
    nchunk = ev_per_tile // CHUNK
    mesh = plsc.VectorSubcoreMesh(core_axis_name="c", subcore_axis_name="s")

    @functools.partial(
        pl.kernel,
        mesh=mesh,
        out_type=jax.ShapeDtypeStruct((NC, 2, NPIX), jnp.float32),
        scratch_types=[
            pltpu.VMEM((CHUNK,), jnp.float32),
            pltpu.VMEM((CHUNK,), jnp.float32),
            pltpu.VMEM((CHUNK,), jnp.float32),
            pltpu.VMEM((CHUNK,), jnp.float32),
            pltpu.VMEM((CHUNK,), jnp.float32),
            pltpu.VMEM((CHUNK,), jnp.int32),
            pltpu.VMEM((CHUNK,), jnp.float32),
            pltpu.VMEM((CHUNK,), jnp.float32),
            pltpu.VMEM((4 * CHUNK,), jnp.int32),
            pltpu.VMEM((4 * CHUNK,), jnp.float32),
            pltpu.VMEM((4 * CHUNK,), jnp.float32),
            pltpu.VMEM_SHARED((NPIX,), jnp.float32),
            pltpu.VMEM_SHARED((NPIX,), jnp.float32),
            pltpu.VMEM_SHARED((NPIX,), jnp.float32),
            pltpu.VMEM_SHARED((NPIX,), jnp.float32),
            pltpu.VMEM_SHARED((NS, 5 * CHUNK), jnp.float32),
        ],
    )
    def body(t_hbm, y_hbm, x_hbm, p0_hbm, p1_hbm, fx_hbm, fy_hbm, z_hbm,
             out_hbm, tbuf, ybuf, xbuf, p0buf, p1buf, fidx, fxb, fyb,
             pidx, valp, valn, spimgp, spimgn, spfx, spfy, spcols):
        c = lax.axis_index("c")
        s = lax.axis_index("s")
        wid = s * NC + c

        pltpu.sync_copy(z_hbm.at[pl.ds(s * ROWS_PER_TILE, ROWS_PER_TILE)],
                        spimgp.at[pl.ds(s * ROWS_PER_TILE, ROWS_PER_TILE)])
        pltpu.sync_copy(z_hbm.at[pl.ds(s * ROWS_PER_TILE, ROWS_PER_TILE)],
                        spimgn.at[pl.ds(s * ROWS_PER_TILE, ROWS_PER_TILE)])
        pltpu.sync_copy(fx_hbm.at[pl.ds(s * ROWS_PER_TILE, ROWS_PER_TILE)],
                        spfx.at[pl.ds(s * ROWS_PER_TILE, ROWS_PER_TILE)])
        pltpu.sync_copy(fy_hbm.at[pl.ds(s * ROWS_PER_TILE, ROWS_PER_TILE)],
                        spfy.at[pl.ds(s * ROWS_PER_TILE, ROWS_PER_TILE)])
        plsc.subcore_barrier()

        def chunk_body(ci, carry):
            base = wid * ev_per_tile + ci * CHUNK
            pltpu.sync_copy(t_hbm.at[pl.ds(base, CHUNK)],
                            spcols.at[s, pl.ds(0 * CHUNK, CHUNK)])
            pltpu.sync_copy(y_hbm.at[pl.ds(base, CHUNK)],
                            spcols.at[s, pl.ds(1 * CHUNK, CHUNK)])
            pltpu.sync_copy(x_hbm.at[pl.ds(base, CHUNK)],
                            spcols.at[s, pl.ds(2 * CHUNK, CHUNK)])
            pltpu.sync_copy(p0_hbm.at[pl.ds(base, CHUNK)],
                            spcols.at[s, pl.ds(3 * CHUNK, CHUNK)])
            pltpu.sync_copy(p1_hbm.at[pl.ds(base, CHUNK)],
                            spcols.at[s, pl.ds(4 * CHUNK, CHUNK)])
            pltpu.sync_copy(spcols.at[s, pl.ds(0 * CHUNK, CHUNK)], tbuf)
            pltpu.sync_copy(spcols.at[s, pl.ds(1 * CHUNK, CHUNK)], ybuf)
            pltpu.sync_copy(spcols.at[s, pl.ds(2 * CHUNK, CHUNK)], xbuf)
            pltpu.sync_copy(spcols.at[s, pl.ds(3 * CHUNK, CHUNK)], p0buf)
            pltpu.sync_copy(spcols.at[s, pl.ds(4 * CHUNK, CHUNK)], p1buf)

            @plsc.parallel_loop(0, NGRP, unroll=4)
            def pass1(g):
                off = g * 16
                y = ybuf[pl.ds(off, 16)]
                x = xbuf[pl.ds(off, 16)]
                ry = _round_half_even_i(y)
                rx = _round_half_even_i(x)
                ry = jnp.minimum(jnp.maximum(ry, 0), H - 1)
                rx = jnp.minimum(jnp.maximum(rx, 0), W - 1)
                fidx[pl.ds(off, 16)] = ry * W + rx

            pltpu.sync_copy(spfx.at[fidx], fxb)
            pltpu.sync_copy(spfy.at[fidx], fyb)

            @plsc.parallel_loop(0, NGRP, unroll=2)
            def pass2(g):
                off = g * 16
                t = tbuf[pl.ds(off, 16)]
                y = ybuf[pl.ds(off, 16)]
                x = xbuf[pl.ds(off, 16)]
                w0 = p0buf[pl.ds(off, 16)]
                w1 = p1buf[pl.ds(off, 16)]
                fx = fxb[pl.ds(off, 16)]
                fy = fyb[pl.ds(off, 16)]
                dt = 1.0 - t
                wy = y + dt * fy
                wx = x + dt * fx
                y0 = _floor_i(wy)
                x0 = _floor_i(wx)
                dy = wy - y0.astype(jnp.float32)
                dx = wx - x0.astype(jnp.float32)
                wty = 1.0 - dy
                wlx = 1.0 - dx
                y1 = y0 + 1
                x1 = x0 + 1
                y0ok = (y0 >= 0) & (y0 < H)
                y1ok = (y1 >= 0) & (y1 < H)
                x0ok = (x0 >= 0) & (x0 < W)
                x1ok = (x1 >= 0) & (x1 < W)
                base = y0 * W + x0
                corners = (
                    (y0ok & x0ok, base, wty * wlx),
                    (y0ok & x1ok, base + 1, wty * dx),
                    (y1ok & x0ok, base + W, dy * wlx),
                    (y1ok & x1ok, base + W + 1, dy * dx),
                )
                for j, (inb, praw, wgt) in enumerate(corners):
                    p = jnp.where(inb, praw, 0)
                    wv = jnp.where(inb, wgt, 0.0)
                    pidx[pl.ds(j * CHUNK + off, 16)] = p
                    valp[pl.ds(j * CHUNK + off, 16)] = wv * w0
                    valn[pl.ds(j * CHUNK + off, 16)] = wv * w1

            pltpu.sync_copy(valp, spimgp.at[pidx], add=True)
            pltpu.sync_copy(valn, spimgn.at[pidx], add=True)
            return carry

        lax.fori_loop(0, nchunk, chunk_body, 0)
        plsc.subcore_barrier()

        pltpu.sync_copy(spimgp.at[pl.ds(s * ROWS_PER_TILE, ROWS_PER_TILE)],
                        out_hbm.at[c, 0, pl.ds(s * ROWS_PER_TILE, ROWS_PER_TILE)])
        pltpu.sync_copy(spimgn.at[pl.ds(s * ROWS_PER_TILE, ROWS_PER_TILE)],
                        out_hbm.at[c, 1, pl.ds(s * ROWS_PER_TILE, ROWS_PER_TILE)])

    return body


def kernel(event_list, flow, pol_mask, event_mask):
    n = event_list.shape[1]
    n_pad = ((n + NW * CHUNK - 1) // (NW * CHUNK)) * (NW * CHUNK)
    pad = n_pad - n
    ev = event_list[0]
    t = ev[:, 0]
    y = ev[:, 1]
    x = ev[:, 2]
    p0 = pol_mask[0, :, 0]
    p1 = pol_mask[0, :, 1]
    if pad:
        z = jnp.zeros((pad,), jnp.float32)
        t = jnp.concatenate([t, z])
        y = jnp.concatenate([y, z])
        x = jnp.concatenate([x, z])
        p0 = jnp.concatenate([p0, z])
        p1 = jnp.concatenate([p1, z])
    fx = flow[0, 0].reshape(-1)
    fy = flow[0, 1].reshape(-1)
    zeros1 = jnp.zeros((NPIX,), jnp.float32)
    out = _make_sc_kernel(n_pad)(t, y, x, p0, p1, fx, fy, zeros1)
    acc = out[0] + out[1]
    return acc.reshape(1, 2, H, W)
